# Initial kernel scaffold; baseline (speedup 1.0000x reference)
import jax
import jax.numpy as jnp
from jax.experimental import pallas as pl


def kernel(obs, edge_index, W1, b1, Wg, bg, W3, b3, Wh, bh, Wv, bv):
    raise NotImplementedError("write your pallas kernel here")


# speedup vs baseline: 2.9086x; 2.9086x over previous
import functools

import jax
import jax.numpy as jnp
from jax import lax
from jax.experimental import pallas as pl
from jax.experimental.pallas import tpu as pltpu
from jax.experimental.pallas import tpu_sc as plsc

N = 10000
E = 160000
OBS = 256
HID = 512
NACT = 32

NW = 32
EPW = E // NW
NVEC = (EPW + 15) // 16
EBUF = NVEC * 16
NPAD = 10240
NBIN = 32
BINROWS = NPAD // NBIN
BINCAP = 5136
BB = 32
SAFE_DST = 10100
DEGPAD = NPAD
RB = 640
GRID = 16

_scmesh = plsc.VectorSubcoreMesh(core_axis_name="c", subcore_axis_name="s")
_scparams = pltpu.CompilerParams(needs_layout_passes=False)


@functools.partial(
    pl.kernel,
    out_type=(
        jax.ShapeDtypeStruct((NW * NBIN * BINCAP,), jnp.int32),
        jax.ShapeDtypeStruct((NW * NBIN * BINCAP,), jnp.int32),
        jax.ShapeDtypeStruct((NW, NBIN), jnp.int32),
        jax.ShapeDtypeStruct((NW, DEGPAD), jnp.float32),
    ),
    mesh=_scmesh,
    scratch_types=[
        pltpu.VMEM((EBUF,), jnp.int32),
        pltpu.VMEM((EBUF,), jnp.int32),
        pltpu.VMEM((BINCAP,), jnp.int32),
        pltpu.VMEM((BINCAP,), jnp.int32),
        pltpu.VMEM((DEGPAD,), jnp.float32),
        pltpu.VMEM((NBIN,), jnp.int32),
    ],
    compiler_params=_scparams,
)
def _edge_sc(src_hbm, dst_hbm, comp_src, comp_dst, counts, pdeg,
             src_v, dst_v, csrc_v, cdst_v, ldeg, cnt_v):
    c = lax.axis_index("c")
    s = lax.axis_index("s")
    w = s * 2 + c
    e0 = w * EPW
    pltpu.sync_copy(src_hbm.at[pl.ds(e0, EPW)], src_v.at[pl.ds(0, EPW)])
    pltpu.sync_copy(dst_hbm.at[pl.ds(e0, EPW)], dst_v.at[pl.ds(0, EPW)])

    lane = lax.iota(jnp.int32, 16)
    msk8 = lane < (EPW - (NVEC - 1) * 16)
    tail = (NVEC - 1) * 16
    t_s = src_v[pl.ds(tail, 16)]
    t_d = dst_v[pl.ds(tail, 16)]
    src_v[pl.ds(tail, 16)] = jnp.where(msk8, t_s, 0)
    dst_v[pl.ds(tail, 16)] = jnp.where(msk8, t_d, SAFE_DST)

    zf = jnp.zeros((16,), jnp.float32)
    def _zero(i, carry):
        ldeg[pl.ds(i * 16, 16)] = zf
        return carry
    lax.fori_loop(0, DEGPAD // 16, _zero, 0)

    onesf = jnp.ones((16,), jnp.float32)
    def _deg(i, carry):
        dvec = dst_v[pl.ds(i * 16, 16)]
        plsc.addupdate_scatter(ldeg, [dvec], onesf)
        return carry
    lax.fori_loop(0, NVEC, _deg, 0)
    pltpu.sync_copy(ldeg, pdeg.at[w])

    zi = jnp.zeros((16,), jnp.int32)
    padd = jnp.full((16,), NPAD - 8, jnp.int32)
    cnt_lo = zi
    cnt_hi = zi
    for k in range(NBIN):
        lo = k * BINROWS
        hi = lo + BINROWS
        def _bin(i, cnt):
            sv = src_v[pl.ds(i * 16, 16)]
            dv = dst_v[pl.ds(i * 16, 16)]
            m = (dv >= lo) & (dv < hi)
            plsc.store_compressed(csrc_v.at[pl.ds(cnt, 16)], sv, mask=m)
            plsc.store_compressed(cdst_v.at[pl.ds(cnt, 16)], dv, mask=m)
            return cnt + jnp.sum(m.astype(jnp.int32))
        cnt = lax.fori_loop(0, NVEC, _bin, jnp.int32(0))
        for j in range(BB // 16):
            csrc_v[pl.ds(cnt + j * 16, 16)] = zi
            cdst_v[pl.ds(cnt + j * 16, 16)] = padd
        pltpu.sync_copy(csrc_v, comp_src.at[pl.ds((w * NBIN + k) * BINCAP, BINCAP)])
        pltpu.sync_copy(cdst_v, comp_dst.at[pl.ds((w * NBIN + k) * BINCAP, BINCAP)])
        if k < 16:
            cnt_lo = jnp.where(lane == k, cnt, cnt_lo)
        else:
            cnt_hi = jnp.where(lane == (k - 16), cnt, cnt_hi)
    cnt_v[pl.ds(0, 16)] = cnt_lo
    cnt_v[pl.ds(16, 16)] = cnt_hi
    pltpu.sync_copy(cnt_v, counts.at[w])


FINE = 160
FACC = FINE + 8


@functools.partial(
    pl.kernel,
    out_type=jax.ShapeDtypeStruct((NPAD, HID), jnp.float32),
    mesh=_scmesh,
    scratch_types=[
        pltpu.VMEM((BB,), jnp.int32),
        pltpu.VMEM((BB,), jnp.int32),
        pltpu.VMEM((BINCAP,), jnp.int32),
        pltpu.VMEM((BINCAP,), jnp.int32),
        pltpu.VMEM((BB, HID), jnp.float32),
        pltpu.VMEM((FACC, HID), jnp.float32),
        pltpu.VMEM((NW, NBIN), jnp.int32),
        pltpu.SemaphoreType.DMA,
    ],
    compiler_params=_scparams,
)
def _seg_sc(y, comp_src, comp_dst, counts, z,
            bsrc, bdst, fsrc, fdst, rows, acc, cnts_all, sem):
    c = lax.axis_index("c")
    s = lax.axis_index("s")
    w = s * 2 + c
    lane = lax.iota(jnp.int32, 16)

    pltpu.sync_copy(counts, cnts_all)

    wbase = w * BINROWS
    khalf = (w // 16) * 16
    ksel = w - khalf
    zf = jnp.zeros((16,), jnp.float32)
    zi = jnp.zeros((16,), jnp.int32)
    dumpv = jnp.full((16,), FINE, jnp.int32)

    for p in range(2):
        plo = wbase + p * FINE
        phi = plo + FINE

        def _zacc(i, carry):
            acc[i // (HID // 16), pl.ds((i % (HID // 16)) * 16, 16)] = zf
            return carry
        lax.fori_loop(0, FACC * (HID // 16), _zacc, 0)

        def _list(wp, carry):
            cvec = cnts_all[wp, pl.ds(khalf, 16)]
            n = jnp.sum(jnp.where(lane == ksel, cvec, 0))
            nb = (n + (BB - 1)) // BB
            lbase = (wp * NBIN + w) * BINCAP

            def _scan(b, mc):
                off = b * BB
                pltpu.sync_copy(comp_src.at[pl.ds(lbase + off, BB)], bsrc)
                pltpu.sync_copy(comp_dst.at[pl.ds(lbase + off, BB)], bdst)
                for vv in range(BB // 16):
                    sv = bsrc[pl.ds(vv * 16, 16)]
                    dv = bdst[pl.ds(vv * 16, 16)]
                    m = (dv >= plo) & (dv < phi)
                    plsc.store_compressed(fsrc.at[pl.ds(mc, 16)], sv, mask=m)
                    plsc.store_compressed(fdst.at[pl.ds(mc, 16)], dv - plo,
                                          mask=m)
                    mc = mc + jnp.sum(m.astype(jnp.int32))
                return mc
            mc = lax.fori_loop(0, nb, _scan, jnp.int32(0))

            for j in range(BB // 16):
                fsrc[pl.ds(mc + j * 16, 16)] = zi
                fdst[pl.ds(mc + j * 16, 16)] = dumpv
            nb2 = (mc + (BB - 1)) // BB

            def _accum(b, carry2):
                off = b * BB
                pltpu.async_copy(y.at[fsrc.at[pl.ds(off, BB)]], rows,
                                 sem).wait()
                for g in range(BB // 16):
                    dvec = fdst[pl.ds(off + g * 16, 16)]
                    for l in range(16):
                        d = dvec[l]
                        r = g * 16 + l
                        for jj in range(HID // 16):
                            plsc.addupdate(acc.at[d, pl.ds(jj * 16, 16)],
                                           rows[r, pl.ds(jj * 16, 16)])
                return carry2
            lax.fori_loop(0, nb2, _accum, 0)
            return carry
        lax.fori_loop(0, NW, _list, 0)

        pltpu.sync_copy(acc.at[pl.ds(0, FINE)], z.at[pl.ds(plo, FINE)])


def _m1_body(obs_ref, w1_ref, b1_ref, x_ref):
    x_ref[...] = jnp.tanh(
        jnp.dot(obs_ref[...], w1_ref[...], preferred_element_type=jnp.float32)
        + b1_ref[...])


_m1 = pl.pallas_call(
    _m1_body,
    grid=(GRID,),
    in_specs=[
        pl.BlockSpec((RB, OBS), lambda i: (i, 0)),
        pl.BlockSpec((OBS, HID), lambda i: (0, 0)),
        pl.BlockSpec((1, HID), lambda i: (0, 0)),
    ],
    out_specs=pl.BlockSpec((RB, HID), lambda i: (i, 0)),
    out_shape=jax.ShapeDtypeStruct((N, HID), jnp.float32),
)


def _y_body(x_ref, wg_ref, pdeg_ref, y_ref):
    deg = jnp.sum(pdeg_ref[...], axis=0) + 1.0
    dinv = lax.rsqrt(deg)
    y_ref[...] = (
        jnp.dot(x_ref[...], wg_ref[...], preferred_element_type=jnp.float32)
        * dinv[:, None])


_yk = pl.pallas_call(
    _y_body,
    grid=(GRID,),
    in_specs=[
        pl.BlockSpec((RB, HID), lambda i: (i, 0)),
        pl.BlockSpec((HID, HID), lambda i: (0, 0)),
        pl.BlockSpec((NW, RB), lambda i: (0, i)),
    ],
    out_specs=pl.BlockSpec((RB, HID), lambda i: (i, 0)),
    out_shape=jax.ShapeDtypeStruct((N, HID), jnp.float32),
)


def _f_body(x_ref, y_ref, z_ref, pdeg_ref, w3a_ref, w3b_ref, b3_ref,
            wh_ref, bh_ref, wv_ref, bv_ref, bg_ref, a_ref, v_ref):
    deg = jnp.sum(pdeg_ref[...], axis=0) + 1.0
    dinv = lax.rsqrt(deg)[:, None]
    agg = dinv * (z_ref[...] + y_ref[...])
    h = jnp.tanh(agg + bg_ref[...])
    h2 = jnp.tanh(
        jnp.dot(x_ref[...], w3a_ref[...], preferred_element_type=jnp.float32)
        + jnp.dot(h, w3b_ref[...], preferred_element_type=jnp.float32)
        + b3_ref[...])
    lg = jnp.dot(h2, wh_ref[...], preferred_element_type=jnp.float32) + bh_ref[...]
    m = jnp.max(lg, axis=-1, keepdims=True)
    a_ref[...] = lg - (m + jnp.log(jnp.sum(jnp.exp(lg - m), axis=-1,
                                           keepdims=True)))
    v_ref[...] = (jnp.dot(h2, wv_ref[...], preferred_element_type=jnp.float32)
                  + bv_ref[...])


_fk = pl.pallas_call(
    _f_body,
    grid=(GRID,),
    in_specs=[
        pl.BlockSpec((RB, HID), lambda i: (i, 0)),
        pl.BlockSpec((RB, HID), lambda i: (i, 0)),
        pl.BlockSpec((RB, HID), lambda i: (i, 0)),
        pl.BlockSpec((NW, RB), lambda i: (0, i)),
        pl.BlockSpec((HID, HID), lambda i: (0, 0)),
        pl.BlockSpec((HID, HID), lambda i: (0, 0)),
        pl.BlockSpec((1, HID), lambda i: (0, 0)),
        pl.BlockSpec((HID, NACT), lambda i: (0, 0)),
        pl.BlockSpec((1, NACT), lambda i: (0, 0)),
        pl.BlockSpec((HID, 1), lambda i: (0, 0)),
        pl.BlockSpec((1, 1), lambda i: (0, 0)),
        pl.BlockSpec((1, HID), lambda i: (0, 0)),
    ],
    out_specs=[
        pl.BlockSpec((RB, NACT), lambda i: (i, 0)),
        pl.BlockSpec((RB, 1), lambda i: (i, 0)),
    ],
    out_shape=[
        jax.ShapeDtypeStruct((N, NACT), jnp.float32),
        jax.ShapeDtypeStruct((N, 1), jnp.float32),
    ],
)


def kernel(obs, edge_index, W1, b1, Wg, bg, W3, b3, Wh, bh, Wv, bv):
    comp_src, comp_dst, counts, pdeg = _edge_sc(edge_index[0], edge_index[1])
    x = _m1(obs, W1, b1.reshape(1, HID))
    y = _yk(x, Wg, pdeg)
    z = _seg_sc(y, comp_src, comp_dst, counts)
    a, v = _fk(x, y, z, pdeg,
               W3[:HID], W3[HID:], b3.reshape(1, HID),
               Wh, bh.reshape(1, NACT), Wv, bv.reshape(1, 1),
               bg.reshape(1, HID))
    return (a, v)
